# baseline (speedup 1.0000x reference)
import functools
import jax
import jax.numpy as jnp
from jax import lax
from jax.experimental import pallas as pl
from jax.experimental.pallas import tpu as pltpu
from jax.experimental.pallas import tpu_sc as plsc

F_SIMP = 4096
F_ORIG = 8192
NV_ORIG = 6000
NV_SIMP = 3000
VOFF_SIMP = 3 * NV_ORIG
FOFF_SIMP = 3 * F_ORIG
FEAT = 8
JBLK = 1024
CHUNK = 128
NLANE = 16
TOP = -65536
ONE_PAIR = (0x3F80 << 16) | 0x3F80
ONE_LO = 0x3F80
ROW_A = 0
ROW_B = F_SIMP
ROW_NA = F_SIMP + F_ORIG
ROW_P = ROW_NA + F_SIMP
ROWS_OUT = ROW_P + F_SIMP
NVFLAT = 3 * (NV_ORIG + NV_SIMP)
NFFLAT = 3 * (F_ORIG + F_SIMP)


def _rne_hi_bits(x):
    xb = plsc.bitcast(x, jnp.int32)
    rbit = lax.shift_right_logical(xb, 16) & 1
    return (xb + 0x7FFF + rbit) & TOP


def _split(x):
    hb = _rne_hi_bits(x)
    lo = x - plsc.bitcast(hb, jnp.float32)
    lb = _rne_hi_bits(lo)
    return hb, lb


def _pair(ub, vb):
    return lax.shift_right_logical(ub, 16) | (vb & TOP)


OFF_V = NFFLAT
OFF_P = NFFLAT + NVFLAT


def _sc_body(all_ref, out_ref, vtab, fchunk, pchunk, feat, sfeat):
    wid = lax.axis_index("s") * 2 + lax.axis_index("c")
    lane = lax.iota(jnp.int32, NLANE)
    lane3 = lane * 3

    def cvec(c):
        return jnp.full((NLANE,), c, jnp.int32)

    zeros_i = jnp.full((NLANE,), 0, jnp.int32)
    for g in range(CHUNK // NLANE):
        ridx = g * NLANE + lane
        for c in range(1, FEAT):
            plsc.store_scatter(sfeat, [ridx, cvec(c)], zeros_i)

    pltpu.sync_copy(all_ref.at[pl.ds(OFF_V, NVFLAT)], vtab)

    def do_chunk(flat_face_base, vtx_off, out_base, is_a):
        pltpu.sync_copy(all_ref.at[pl.ds(flat_face_base, 3 * CHUNK)], fchunk)
        for g in range(CHUNK // NLANE):
            ridx = g * NLANE + lane
            f3 = 3 * NLANE * g + lane3
            vid0 = plsc.load_gather(fchunk, [f3]) * 3 + vtx_off
            vid1 = plsc.load_gather(fchunk, [f3 + 1]) * 3 + vtx_off
            vid2 = plsc.load_gather(fchunk, [f3 + 2]) * 3 + vtx_off

            def coord(c):
                s = (plsc.bitcast(plsc.load_gather(vtab, [vid0 + c]),
                                  jnp.float32)
                     + plsc.bitcast(plsc.load_gather(vtab, [vid1 + c]),
                                    jnp.float32)
                     + plsc.bitcast(plsc.load_gather(vtab, [vid2 + c]),
                                    jnp.float32))
                return s * (1.0 / 3.0)

            x, y, z = coord(0), coord(1), coord(2)
            n2 = x * x + y * y + z * z
            if is_a:
                hx, lx = _split(-2.0 * x)
                hy, ly = _split(-2.0 * y)
                hz, lz = _split(-2.0 * z)
                cols = [
                    _pair(hx, hx), _pair(lx, hy), _pair(hy, ly),
                    _pair(hz, hz), _pair(lz, lx), _pair(ly, lz),
                    cvec(ONE_PAIR), cvec(ONE_LO),
                ]
                plsc.store_scatter(sfeat, [ridx, cvec(0)],
                                   plsc.bitcast(n2, jnp.int32))
            else:
                hx, lx = _split(x)
                hy, ly = _split(y)
                hz, lz = _split(z)
                nh, nl = _split(n2)
                nr = _rne_hi_bits(n2 - plsc.bitcast(nh, jnp.float32)
                                  - plsc.bitcast(nl, jnp.float32))
                cols = [
                    _pair(hx, lx), _pair(hx, hy), _pair(ly, hy),
                    _pair(hz, lz), _pair(hz, lx), _pair(ly, lz),
                    _pair(nh, nl), _pair(nr, cvec(0)),
                ]
            for c, col in enumerate(cols):
                plsc.store_scatter(feat, [ridx, cvec(c)], col)

        pltpu.sync_copy(feat, out_ref.at[pl.ds(out_base, CHUNK)])
        if is_a:
            pltpu.sync_copy(sfeat, out_ref.at[pl.ds(ROW_NA + out_base, CHUNK)])

    base_a = wid * CHUNK
    do_chunk(FOFF_SIMP + 3 * base_a, VOFF_SIMP, ROW_A + base_a, True)

    pltpu.sync_copy(all_ref.at[pl.ds(OFF_P + base_a, CHUNK)], pchunk)
    for g in range(CHUNK // NLANE):
        ridx = g * NLANE + lane
        v = pchunk[pl.ds(g * NLANE, NLANE)]
        plsc.store_scatter(sfeat, [ridx, cvec(0)], v)
    pltpu.sync_copy(sfeat, out_ref.at[pl.ds(ROW_P + base_a, CHUNK)])

    do_chunk(3 * (wid * 2 * CHUNK), 0, ROW_B + wid * 2 * CHUNK, False)
    do_chunk(3 * ((wid * 2 + 1) * CHUNK), 0, ROW_B + (wid * 2 + 1) * CHUNK,
             False)


def _sc_features(all_bits):
    mesh = plsc.VectorSubcoreMesh(core_axis_name="c", subcore_axis_name="s")
    fn = pl.kernel(
        _sc_body,
        out_type=jax.ShapeDtypeStruct((ROWS_OUT, FEAT), jnp.int32),
        mesh=mesh,
        compiler_params=pltpu.CompilerParams(
            needs_layout_passes=False, use_tc_tiling_on_sc=False),
        scratch_types=[
            pltpu.VMEM((NVFLAT,), jnp.int32),
            pltpu.VMEM((3 * CHUNK,), jnp.int32),
            pltpu.VMEM((CHUNK,), jnp.int32),
            pltpu.VMEM((CHUNK, FEAT), jnp.int32),
            pltpu.VMEM((CHUNK, FEAT), jnp.int32),
        ],
    )
    return fn(all_bits)


def _unpack_bf16(x_i32):
    lo = lax.bitcast_convert_type(lax.shift_left(x_i32, 16), jnp.float32)
    hi = lax.bitcast_convert_type(x_i32 & TOP, jnp.float32)
    return jnp.concatenate([lo.astype(jnp.bfloat16),
                            hi.astype(jnp.bfloat16)], axis=1)


def _tc_body(a_ref, b_ref, na_ref, p_ref, out_ref, abf_ref, acc_ref):
    j = pl.program_id(0)
    nj = pl.num_programs(0)

    @pl.when(j == 0)
    def _():
        abf_ref[...] = _unpack_bf16(a_ref[...])

    b_bf = _unpack_bf16(b_ref[...])
    g = lax.dot_general(
        abf_ref[...], b_bf,
        (((1,), (1,)), ((), ())),
        preferred_element_type=jnp.float32,
    )
    m = jnp.min(g, axis=1, keepdims=True)

    @pl.when(j == 0)
    def _():
        acc_ref[...] = m

    @pl.when(j > 0)
    def _():
        acc_ref[...] = jnp.minimum(acc_ref[...], m)

    @pl.when(j == nj - 1)
    def _():
        na = lax.bitcast_convert_type(na_ref[...][:, 0:1], jnp.float32)
        p = lax.bitcast_convert_type(p_ref[...][:, 0:1], jnp.float32)
        out_ref[...] = jnp.sum((acc_ref[...] + na) * p, keepdims=True)


def _tc_min_loss(packed):
    grid = (F_ORIG // JBLK,)
    nb_blk = F_SIMP // JBLK
    return pl.pallas_call(
        _tc_body,
        grid=grid,
        in_specs=[
            pl.BlockSpec((F_SIMP, FEAT), lambda j: (0, 0)),
            pl.BlockSpec((JBLK, FEAT), lambda j: (nb_blk + j, 0)),
            pl.BlockSpec((F_SIMP, FEAT), lambda j: (ROW_NA // F_SIMP, 0)),
            pl.BlockSpec((F_SIMP, FEAT), lambda j: (ROW_P // F_SIMP, 0)),
        ],
        out_specs=pl.BlockSpec((1, 1), lambda j: (0, 0)),
        out_shape=jax.ShapeDtypeStruct((1, 1), jnp.float32),
        scratch_shapes=[pltpu.VMEM((F_SIMP, 2 * FEAT), jnp.bfloat16),
                        pltpu.VMEM((F_SIMP, 1), jnp.float32)],
    )(packed, packed, packed, packed)


def kernel(original_vertices, original_faces, simplified_vertices,
           simplified_faces, face_probabilities):
    of = original_faces.astype(jnp.int32)
    sf = simplified_faces.astype(jnp.int32)
    all_bits = jnp.concatenate([
        of.reshape(-1), sf.reshape(-1),
        lax.bitcast_convert_type(original_vertices, jnp.int32).reshape(-1),
        lax.bitcast_convert_type(simplified_vertices, jnp.int32).reshape(-1),
        lax.bitcast_convert_type(face_probabilities, jnp.int32),
    ])
    packed = _sc_features(all_bits)
    loss = _tc_min_loss(packed)
    return loss[0, 0]


# speedup vs baseline: 1.0749x; 1.0749x over previous
import functools
import jax
import jax.numpy as jnp
from jax import lax
from jax.experimental import pallas as pl
from jax.experimental.pallas import tpu as pltpu
from jax.experimental.pallas import tpu_sc as plsc

F_SIMP = 4096
F_ORIG = 8192
NV_ORIG = 6000
NV_SIMP = 3000
VOFF_SIMP = 3 * NV_ORIG
FOFF_SIMP = 3 * F_ORIG
FEAT = 8
JBLK = 2048
CHUNK = 128
NLANE = 16
TOP = -65536
ONE_PAIR = (0x3F80 << 16) | 0x3F80
ONE_LO = 0x3F80
ROW_A = 0
ROW_B = F_SIMP
ROW_NA = F_SIMP + F_ORIG
ROW_P = ROW_NA + F_SIMP
ROWS_OUT = ROW_P + F_SIMP
NVFLAT = 3 * (NV_ORIG + NV_SIMP)
NFFLAT = 3 * (F_ORIG + F_SIMP)


def _rne_hi_bits(x):
    xb = plsc.bitcast(x, jnp.int32)
    rbit = lax.shift_right_logical(xb, 16) & 1
    return (xb + 0x7FFF + rbit) & TOP


def _split(x):
    hb = _rne_hi_bits(x)
    lo = x - plsc.bitcast(hb, jnp.float32)
    lb = _rne_hi_bits(lo)
    return hb, lb


def _pair(ub, vb):
    return lax.shift_right_logical(ub, 16) | (vb & TOP)


def _sc_body(vt_ref, ft_ref, p_ref, out_ref, vtab, fchunk, pchunk, feat,
             sfeat):
    wid = lax.axis_index("s") * 2 + lax.axis_index("c")
    lane = lax.iota(jnp.int32, NLANE)
    lane3 = lane * 3

    def cvec(c):
        return jnp.full((NLANE,), c, jnp.int32)

    zeros_i = jnp.full((NLANE,), 0, jnp.int32)
    for g in range(CHUNK // NLANE):
        ridx = g * NLANE + lane
        for c in range(1, FEAT):
            plsc.store_scatter(sfeat, [ridx, cvec(c)], zeros_i)

    pltpu.sync_copy(vt_ref, vtab)

    def do_chunk(flat_face_base, vtx_off, out_base, is_a):
        pltpu.sync_copy(ft_ref.at[pl.ds(flat_face_base, 3 * CHUNK)], fchunk)
        for g in range(CHUNK // NLANE):
            ridx = g * NLANE + lane
            f3 = 3 * NLANE * g + lane3
            vid0 = plsc.load_gather(fchunk, [f3]) * 3 + vtx_off
            vid1 = plsc.load_gather(fchunk, [f3 + 1]) * 3 + vtx_off
            vid2 = plsc.load_gather(fchunk, [f3 + 2]) * 3 + vtx_off

            def coord(c):
                s = (plsc.load_gather(vtab, [vid0 + c])
                     + plsc.load_gather(vtab, [vid1 + c])
                     + plsc.load_gather(vtab, [vid2 + c]))
                return s * (1.0 / 3.0)

            x, y, z = coord(0), coord(1), coord(2)
            n2 = x * x + y * y + z * z
            if is_a:
                hx, lx = _split(-2.0 * x)
                hy, ly = _split(-2.0 * y)
                hz, lz = _split(-2.0 * z)
                cols = [
                    _pair(hx, hx), _pair(lx, hy), _pair(hy, ly),
                    _pair(hz, hz), _pair(lz, lx), _pair(ly, lz),
                    cvec(ONE_PAIR), cvec(ONE_LO),
                ]
                plsc.store_scatter(sfeat, [ridx, cvec(0)],
                                   plsc.bitcast(n2, jnp.int32))
            else:
                hx, lx = _split(x)
                hy, ly = _split(y)
                hz, lz = _split(z)
                nh, nl = _split(n2)
                nr = _rne_hi_bits(n2 - plsc.bitcast(nh, jnp.float32)
                                  - plsc.bitcast(nl, jnp.float32))
                cols = [
                    _pair(hx, lx), _pair(hx, hy), _pair(ly, hy),
                    _pair(hz, lz), _pair(hz, lx), _pair(ly, lz),
                    _pair(nh, nl), _pair(nr, cvec(0)),
                ]
            for c, col in enumerate(cols):
                plsc.store_scatter(feat, [ridx, cvec(c)], col)

        pltpu.sync_copy(feat, out_ref.at[pl.ds(out_base, CHUNK)])
        if is_a:
            pltpu.sync_copy(sfeat, out_ref.at[pl.ds(ROW_NA + out_base, CHUNK)])

    base_a = wid * CHUNK
    do_chunk(FOFF_SIMP + 3 * base_a, VOFF_SIMP, ROW_A + base_a, True)

    pltpu.sync_copy(p_ref.at[pl.ds(base_a, CHUNK)], pchunk)
    for g in range(CHUNK // NLANE):
        ridx = g * NLANE + lane
        v = pchunk[pl.ds(g * NLANE, NLANE)]
        plsc.store_scatter(sfeat, [ridx, cvec(0)], plsc.bitcast(v, jnp.int32))
    pltpu.sync_copy(sfeat, out_ref.at[pl.ds(ROW_P + base_a, CHUNK)])

    do_chunk(3 * (wid * 2 * CHUNK), 0, ROW_B + wid * 2 * CHUNK, False)
    do_chunk(3 * ((wid * 2 + 1) * CHUNK), 0, ROW_B + (wid * 2 + 1) * CHUNK,
             False)


def _sc_features(vflat, fflat, probs):
    mesh = plsc.VectorSubcoreMesh(core_axis_name="c", subcore_axis_name="s")
    fn = pl.kernel(
        _sc_body,
        out_type=jax.ShapeDtypeStruct((ROWS_OUT, FEAT), jnp.int32),
        mesh=mesh,
        compiler_params=pltpu.CompilerParams(
            needs_layout_passes=False, use_tc_tiling_on_sc=False),
        scratch_types=[
            pltpu.VMEM((NVFLAT,), jnp.float32),
            pltpu.VMEM((3 * CHUNK,), jnp.int32),
            pltpu.VMEM((CHUNK,), jnp.float32),
            pltpu.VMEM((CHUNK, FEAT), jnp.int32),
            pltpu.VMEM((CHUNK, FEAT), jnp.int32),
        ],
    )
    return fn(vflat, fflat, probs)


def _unpack_bf16(x_i32):
    lo = lax.bitcast_convert_type(lax.shift_left(x_i32, 16), jnp.float32)
    hi = lax.bitcast_convert_type(x_i32 & TOP, jnp.float32)
    return jnp.concatenate([lo.astype(jnp.bfloat16),
                            hi.astype(jnp.bfloat16)], axis=1)


def _tc_body(a_ref, b_ref, na_ref, p_ref, out_ref, abf_ref, acc_ref):
    j = pl.program_id(0)
    nj = pl.num_programs(0)

    @pl.when(j == 0)
    def _():
        abf_ref[...] = _unpack_bf16(a_ref[...])

    b_bf = _unpack_bf16(b_ref[...])
    g = lax.dot_general(
        abf_ref[...], b_bf,
        (((1,), (1,)), ((), ())),
        preferred_element_type=jnp.float32,
    )
    m = jnp.min(g, axis=1, keepdims=True)

    @pl.when(j == 0)
    def _():
        acc_ref[...] = m

    @pl.when(j > 0)
    def _():
        acc_ref[...] = jnp.minimum(acc_ref[...], m)

    @pl.when(j == nj - 1)
    def _():
        na = lax.bitcast_convert_type(na_ref[...][:, 0:1], jnp.float32)
        p = lax.bitcast_convert_type(p_ref[...][:, 0:1], jnp.float32)
        out_ref[...] = jnp.sum((acc_ref[...] + na) * p, keepdims=True)


def _tc_min_loss(packed):
    grid = (F_ORIG // JBLK,)
    nb_blk = F_SIMP // JBLK
    return pl.pallas_call(
        _tc_body,
        grid=grid,
        in_specs=[
            pl.BlockSpec((F_SIMP, FEAT), lambda j: (0, 0)),
            pl.BlockSpec((JBLK, FEAT), lambda j: (nb_blk + j, 0)),
            pl.BlockSpec((F_SIMP, FEAT), lambda j: (ROW_NA // F_SIMP, 0)),
            pl.BlockSpec((F_SIMP, FEAT), lambda j: (ROW_P // F_SIMP, 0)),
        ],
        out_specs=pl.BlockSpec((1, 1), lambda j: (0, 0)),
        out_shape=jax.ShapeDtypeStruct((1, 1), jnp.float32),
        scratch_shapes=[pltpu.VMEM((F_SIMP, 2 * FEAT), jnp.bfloat16),
                        pltpu.VMEM((F_SIMP, 1), jnp.float32)],
    )(packed, packed, packed, packed)


def kernel(original_vertices, original_faces, simplified_vertices,
           simplified_faces, face_probabilities):
    of = original_faces.astype(jnp.int32)
    sf = simplified_faces.astype(jnp.int32)
    vflat = jnp.concatenate([original_vertices.reshape(-1),
                             simplified_vertices.reshape(-1)])
    fflat = jnp.concatenate([of.reshape(-1), sf.reshape(-1)])
    packed = _sc_features(vflat, fflat, face_probabilities)
    loss = _tc_min_loss(packed)
    return loss[0, 0]
